# baseline (speedup 1.0000x reference)
import functools

import jax
import jax.numpy as jnp
from jax import lax
from jax.experimental import pallas as pl
from jax.experimental.pallas import tpu as pltpu
from jax.experimental.pallas import tpu_sc as plsc

D = 512
NC = 1
NS = 16
NW = NC * NS
ROWS = D // NW

_mesh = plsc.VectorSubcoreMesh(
    core_axis_name="c", subcore_axis_name="s", num_cores=NC, num_subcores=NS
)


@functools.partial(
    pl.kernel,
    mesh=_mesh,
    out_type=(
        jax.ShapeDtypeStruct((D, D), jnp.int32),
        jax.ShapeDtypeStruct((1, 3), jnp.float32),
        jax.ShapeDtypeStruct((1, 1), jnp.float32),
        jax.ShapeDtypeStruct((1, 2), jnp.int32),
    ),
    scratch_types=[
        pltpu.VMEM((ROWS, D), jnp.int32),
        pltpu.VMEM((16,), jnp.int32),
        pltpu.VMEM((16,), jnp.float32),
        pltpu.VMEM((16,), jnp.float32),
        pltpu.VMEM((16,), jnp.int32),
        pltpu.SemaphoreType.DMA,
        [pltpu.SemaphoreType.DMA] * 4,
        [pltpu.SemaphoreType.DMA] * 4,
    ],
    compiler_params=pltpu.CompilerParams(needs_layout_passes=False),
)
def _graphgen_sc(ev_hbm, mat_hbm, out_hbm, nodes_hbm, feat_hbm, edges_hbm,
                 slab_v, ev_v, aux_v, feat_v, zed_v, sem_ev, sems_in, sems_out):
    wid = lax.axis_index("s") * NC + lax.axis_index("c")
    base = wid * ROWS
    CH = 4
    CR = ROWS // CH

    cp_ev = pltpu.async_copy(ev_hbm, ev_v, sem_ev)
    cps_in = [
        pltpu.async_copy(
            mat_hbm.at[pl.ds(base + i * CR, CR)],
            slab_v.at[pl.ds(i * CR, CR)],
            sems_in[i],
        )
        for i in range(CH)
    ]

    cp_ev.wait()
    lane = lax.iota(jnp.int32, 16)
    zero = jnp.zeros((16,), jnp.int32)
    ev = ev_v[...]
    x_s = jnp.sum(jnp.where(lane == 0, ev, zero), dtype=jnp.int32)
    y_s = jnp.sum(jnp.where(lane == 1, ev, zero), dtype=jnp.int32)

    cps_out = []
    for i in range(CH):
        cps_in[i].wait()
        lo = base + i * CR
        own = (lane == 0) & (x_s >= lo) & (x_s < lo + CR)
        plsc.store_scatter(
            slab_v.at[pl.ds(i * CR, CR)],
            [zero + (x_s - lo), zero + y_s],
            zero,
            mask=own,
        )
        cps_out.append(
            pltpu.async_copy(
                slab_v.at[pl.ds(i * CR, CR)],
                out_hbm.at[pl.ds(lo, CR)],
                sems_out[i],
            )
        )
    for cp in cps_out:
        cp.wait()

    @pl.when(wid == 0)
    def _():
        evf = ev.astype(jnp.float32)
        f_s = jnp.sum(jnp.where(lane == 3, evf, jnp.zeros((16,), jnp.float32)))
        aux_v[...] = evf
        feat_v[...] = jnp.zeros((16,), jnp.float32) + f_s
        zed_v[...] = zero
        i0 = jnp.int32(0)
        pltpu.sync_copy(aux_v.at[pl.ds(0, 3)], nodes_hbm.at[i0])
        pltpu.sync_copy(feat_v.at[pl.ds(0, 1)], feat_hbm.at[i0])
        pltpu.sync_copy(zed_v.at[pl.ds(0, 2)], edges_hbm.at[i0])


def kernel(event, neighbour_matrix):
    ev16 = jnp.zeros((16,), jnp.int32).at[:4].set(event.astype(jnp.int32))
    new_matrix, nodes, features, edges = _graphgen_sc(ev16, neighbour_matrix)
    return nodes, features, edges, new_matrix


# speedup vs baseline: 5.0015x; 5.0015x over previous
import jax
import jax.numpy as jnp
from jax import lax
from jax.experimental import pallas as pl
from jax.experimental.pallas import tpu as pltpu

D = 512


def _tc_body(ev_ref, mat_ref, out_ref, nodes_ref, feat_ref, edges_ref):
    x = ev_ref[0]
    y = ev_ref[1]
    r = lax.broadcasted_iota(jnp.int32, (D, D), 0)
    c = lax.broadcasted_iota(jnp.int32, (D, D), 1)
    out_ref[...] = jnp.where((r == x) & (c == y), 0, mat_ref[...])
    col = lax.broadcasted_iota(jnp.int32, (1, 3), 1)
    xf = x.astype(jnp.float32)
    yf = ev_ref[1].astype(jnp.float32)
    tf = ev_ref[2].astype(jnp.float32)
    nodes_ref[...] = jnp.where(col == 0, xf, jnp.where(col == 1, yf, tf))
    feat_ref[...] = jnp.full((1, 1), ev_ref[3], jnp.float32)
    edges_ref[...] = jnp.zeros((1, 2), jnp.int32)


_tc_call = pl.pallas_call(
    _tc_body,
    in_specs=[
        pl.BlockSpec(memory_space=pltpu.SMEM),
        pl.BlockSpec(memory_space=pltpu.VMEM),
    ],
    out_specs=[
        pl.BlockSpec(memory_space=pltpu.VMEM),
        pl.BlockSpec(memory_space=pltpu.VMEM),
        pl.BlockSpec(memory_space=pltpu.VMEM),
        pl.BlockSpec(memory_space=pltpu.VMEM),
    ],
    out_shape=(
        jax.ShapeDtypeStruct((D, D), jnp.int32),
        jax.ShapeDtypeStruct((1, 3), jnp.float32),
        jax.ShapeDtypeStruct((1, 1), jnp.float32),
        jax.ShapeDtypeStruct((1, 2), jnp.int32),
    ),
)


def kernel(event, neighbour_matrix):
    ev = event.astype(jnp.int32)
    new_matrix, nodes, features, edges = _tc_call(ev, neighbour_matrix)
    return nodes, features, edges, new_matrix
